# Initial kernel scaffold; baseline (speedup 1.0000x reference)
import jax
import jax.numpy as jnp
from jax.experimental import pallas as pl


def kernel(X, W_w, W_b, att_v_user, att_v_poi, att_v_class, att_v_time, att_e_friend, att_e_visit, att_e_occurrence, att_e_self, vertex, edges, E_class_index_0, E_class_index_1, E_class_index_2, E_class_index_3, E_class_index, V_class_index_0, V_class_index_1, V_class_index_2, V_class_index_3, V_class_index):
    raise NotImplementedError("write your pallas kernel here")


# speedup vs baseline: 1.2315x; 1.2315x over previous
import functools

import jax
import jax.numpy as jnp
import numpy as np
from jax.experimental import pallas as pl

N = 50000
NNZ = 800000
EDGE_NUM = 25000
IN = 64
H = 8
C = 8
Q = NNZ // 4


def _build_A(att4):
    eye = jnp.eye(H, dtype=att4.dtype)
    A = att4[:, :, :, None] * eye[None, :, None, :]
    return A.transpose(1, 2, 0, 3).reshape(H * C, 4 * H)


def _mm_body(x_ref, w_ref, b_ref, ae_ref, x0_ref, sv_ref):
    x0 = jnp.dot(x_ref[...], w_ref[...], preferred_element_type=jnp.float32)
    x0 = x0 + b_ref[...]
    x0_ref[...] = x0
    sv_ref[...] = jnp.dot(x0, ae_ref[...], preferred_element_type=jnp.float32)


def _proj(X, Wt, b, A):
    n = X.shape[0]
    BLK = 1000
    grid = (n // BLK,)
    return pl.pallas_call(
        _mm_body,
        grid=grid,
        in_specs=[
            pl.BlockSpec((BLK, IN), lambda i: (i, 0)),
            pl.BlockSpec((IN, H * C), lambda i: (0, 0)),
            pl.BlockSpec((1, H * C), lambda i: (0, 0)),
            pl.BlockSpec((H * C, 4 * H), lambda i: (0, 0)),
        ],
        out_specs=[
            pl.BlockSpec((BLK, H * C), lambda i: (i, 0)),
            pl.BlockSpec((BLK, 4 * H), lambda i: (i, 0)),
        ],
        out_shape=[
            jax.ShapeDtypeStruct((n, H * C), jnp.float32),
            jax.ShapeDtypeStruct((n, 4 * H), jnp.float32),
        ],
    )(X, Wt, b.reshape(1, H * C), A)


def kernel(X, W_w, W_b, att_v_user, att_v_poi, att_v_class, att_v_time,
           att_e_friend, att_e_visit, att_e_occurrence, att_e_self,
           vertex, edges,
           E_class_index_0, E_class_index_1, E_class_index_2, E_class_index_3, E_class_index,
           V_class_index_0, V_class_index_1, V_class_index_2, V_class_index_3, V_class_index):
    att_e = jnp.stack([att_e_friend[0], att_e_visit[0], att_e_occurrence[0], att_e_self[0]])
    att_v = jnp.stack([att_v_user[0], att_v_poi[0], att_v_class[0], att_v_time[0]])
    A_e = _build_A(att_e)
    A_v = _build_A(att_v)

    X0, SvE = _proj(X, W_w.T, W_b, A_e)
    SvE = SvE.reshape(N * 4, H)

    cls_e = E_class_index[:, 0] // Q
    cls_v = V_class_index[:, 0] // Q
    rE = vertex * 4 + cls_e
    rV = edges * 4 + cls_v

    s = SvE[rE]
    g = jnp.exp(jnp.where(s >= 0, s, 0.2 * s))
    denomE = jax.ops.segment_sum(g, edges, num_segments=EDGE_NUM)
    beta = g / (denomE[edges] + 1e-16)
    Xh = X0.reshape(N, H, C)
    Xe = jax.ops.segment_sum(Xh[vertex] * beta[:, :, None], edges,
                             num_segments=EDGE_NUM)

    Xe2 = Xe.reshape(EDGE_NUM, H * C)
    SeV = (Xe2 @ A_v).reshape(EDGE_NUM * 4, H)
    s2 = SeV[rV]
    a = jnp.exp(jnp.where(s2 >= 0, s2, 0.2 * s2))
    denomV = jax.ops.segment_sum(a, vertex, num_segments=N)
    alpha = a / (denomV[vertex] + 1e-16)
    Xv = jax.ops.segment_sum(Xe[edges] * alpha[:, :, None], vertex, num_segments=N)
    return jax.nn.relu(Xv.reshape(N, H * C))


# speedup vs baseline: 1.4450x; 1.1734x over previous
import functools

import jax
import jax.numpy as jnp
import numpy as np
from jax import lax
from jax.experimental import pallas as pl
from jax.experimental.pallas import tpu as pltpu
from jax.experimental.pallas import tpu_sc as plsc

N = 50000
NNZ = 800000
EDGE_NUM = 25000
IN = 64
H = 8
C = 8
Q = NNZ // 4

_NC = 2
_NS = 16
_NW = _NC * _NS


def _build_A16(att4):
    eyeH16 = jnp.eye(16, dtype=att4.dtype)[:H]
    A = att4[:, :, :, None] * eyeH16[None, :, None, :]
    return A.transpose(1, 2, 0, 3).reshape(H * C, 4 * 16)


def _mm_body(x_ref, w_ref, b_ref, ae_ref, x0_ref, sv_ref):
    x0 = jnp.dot(x_ref[...], w_ref[...], preferred_element_type=jnp.float32)
    x0 = x0 + b_ref[...]
    x0_ref[...] = x0
    sv_ref[...] = jnp.dot(x0, ae_ref[...], preferred_element_type=jnp.float32)


def _proj(X, Wt, b, A):
    n = X.shape[0]
    BLK = 1000
    return pl.pallas_call(
        _mm_body,
        grid=(n // BLK,),
        in_specs=[
            pl.BlockSpec((BLK, IN), lambda i: (i, 0)),
            pl.BlockSpec((IN, H * C), lambda i: (0, 0)),
            pl.BlockSpec((1, H * C), lambda i: (0, 0)),
            pl.BlockSpec((H * C, 64), lambda i: (0, 0)),
        ],
        out_specs=[
            pl.BlockSpec((BLK, H * C), lambda i: (i, 0)),
            pl.BlockSpec((BLK, 64), lambda i: (i, 0)),
        ],
        out_shape=[
            jax.ShapeDtypeStruct((n, H * C), jnp.float32),
            jax.ShapeDtypeStruct((n, 64), jnp.float32),
        ],
    )(X, Wt, b.reshape(1, H * C), A)


def _sc_gather(table, idx, B=2000):
    M = idx.shape[0]
    D = table.shape[1]
    per_w = M // _NW
    nch = per_w // B
    mesh = plsc.VectorSubcoreMesh(core_axis_name="c", subcore_axis_name="s")

    @functools.partial(
        pl.kernel, mesh=mesh,
        out_type=jax.ShapeDtypeStruct((M, D), jnp.float32),
        compiler_params=pltpu.CompilerParams(use_tc_tiling_on_sc=False),
        scratch_types=[
            pltpu.VMEM((B,), jnp.int32),
            pltpu.VMEM((B, D), jnp.float32),
            pltpu.SemaphoreType.DMA,
        ],
    )
    def k(table_hbm, idx_hbm, out_hbm, idx_v, rows_v, sem):
        wid = lax.axis_index("s") * _NC + lax.axis_index("c")
        base = wid * per_w

        def body(j, carry):
            off = base + j * B
            pltpu.sync_copy(idx_hbm.at[pl.ds(off, B)], idx_v)
            pltpu.async_copy(table_hbm.at[idx_v], rows_v, sem).wait()
            pltpu.sync_copy(rows_v, out_hbm.at[pl.ds(off, B)])
            return carry

        lax.fori_loop(0, nch, body, 0)

    return k(table, idx)


def kernel(X, W_w, W_b, att_v_user, att_v_poi, att_v_class, att_v_time,
           att_e_friend, att_e_visit, att_e_occurrence, att_e_self,
           vertex, edges,
           E_class_index_0, E_class_index_1, E_class_index_2, E_class_index_3, E_class_index,
           V_class_index_0, V_class_index_1, V_class_index_2, V_class_index_3, V_class_index):
    att_e = jnp.stack([att_e_friend[0], att_e_visit[0], att_e_occurrence[0], att_e_self[0]])
    att_v = jnp.stack([att_v_user[0], att_v_poi[0], att_v_class[0], att_v_time[0]])
    A_e = _build_A16(att_e)
    A_v = _build_A16(att_v)

    X0, SvE = _proj(X, W_w.T, W_b, A_e)
    SvE = SvE.reshape(N * 4, 16)

    cls_e = E_class_index[:, 0] // Q
    cls_v = V_class_index[:, 0] // Q
    rE = vertex * 4 + cls_e
    rV = edges * 4 + cls_v

    s = _sc_gather(SvE, rE)[:, :H]
    g = jnp.exp(jnp.where(s >= 0, s, 0.2 * s))
    denomE = jax.ops.segment_sum(g, edges, num_segments=EDGE_NUM)
    beta = g / (denomE[edges] + 1e-16)
    Xve = _sc_gather(X0, vertex).reshape(NNZ, H, C)
    Xe = jax.ops.segment_sum(Xve * beta[:, :, None], edges,
                             num_segments=EDGE_NUM)

    Xe2 = Xe.reshape(EDGE_NUM, H * C)
    SeV = (Xe2 @ A_v).reshape(EDGE_NUM * 4, 16)
    s2 = _sc_gather(SeV, rV)[:, :H]
    a = jnp.exp(jnp.where(s2 >= 0, s2, 0.2 * s2))
    denomV = jax.ops.segment_sum(a, vertex, num_segments=N)
    alpha = a / (denomV[vertex] + 1e-16)
    Xeg = _sc_gather(Xe2, edges).reshape(NNZ, H, C)
    Xv = jax.ops.segment_sum(Xeg * alpha[:, :, None], vertex, num_segments=N)
    return jax.nn.relu(Xv.reshape(N, H * C))


# speedup vs baseline: 36.3498x; 25.1554x over previous
import functools

import jax
import jax.numpy as jnp
import numpy as np
from jax import lax
from jax.experimental import pallas as pl
from jax.experimental.pallas import tpu as pltpu
from jax.experimental.pallas import tpu_sc as plsc

N = 50000
NNZ = 800000
EDGE_NUM = 25000
IN = 64
H = 8
C = 8
Q = NNZ // 4

_NC = 2
_NS = 16
_NW = _NC * _NS


def _build_A16(att4):
    eyeH16 = jnp.eye(16, dtype=att4.dtype)[:H]
    A = att4[:, :, :, None] * eyeH16[None, :, None, :]
    return A.transpose(1, 2, 0, 3).reshape(H * C, 4 * 16)


def _mm_body(x_ref, w_ref, b_ref, ae_ref, x0_ref, sv_ref):
    x0 = jnp.dot(x_ref[...], w_ref[...], preferred_element_type=jnp.float32)
    x0 = x0 + b_ref[...]
    x0_ref[...] = x0
    sv_ref[...] = jnp.dot(x0, ae_ref[...], preferred_element_type=jnp.float32)


def _proj(X, Wt, b, A):
    n = X.shape[0]
    BLK = 1000
    return pl.pallas_call(
        _mm_body,
        grid=(n // BLK,),
        in_specs=[
            pl.BlockSpec((BLK, IN), lambda i: (i, 0)),
            pl.BlockSpec((IN, H * C), lambda i: (0, 0)),
            pl.BlockSpec((1, H * C), lambda i: (0, 0)),
            pl.BlockSpec((H * C, 64), lambda i: (0, 0)),
        ],
        out_specs=[
            pl.BlockSpec((BLK, H * C), lambda i: (i, 0)),
            pl.BlockSpec((BLK, 64), lambda i: (i, 0)),
        ],
        out_shape=[
            jax.ShapeDtypeStruct((n, H * C), jnp.float32),
            jax.ShapeDtypeStruct((n, 64), jnp.float32),
        ],
    )(X, Wt, b.reshape(1, H * C), A)


def _sc_gather(table, idx, B=1000):
    M = idx.shape[0]
    D = table.shape[1]
    per_w = M // _NW
    nch = per_w // B
    assert M % _NW == 0 and per_w % B == 0 and B % 8 == 0, (M, B)
    mesh = plsc.VectorSubcoreMesh(core_axis_name="c", subcore_axis_name="s")

    @functools.partial(
        pl.kernel, mesh=mesh,
        out_type=jax.ShapeDtypeStruct((M, D), jnp.float32),
        compiler_params=pltpu.CompilerParams(use_tc_tiling_on_sc=False),
        scratch_types=[
            pltpu.VMEM((B,), jnp.int32),
            pltpu.VMEM((B, D), jnp.float32),
            pltpu.SemaphoreType.DMA,
        ],
    )
    def k(table_hbm, idx_hbm, out_hbm, idx_v, rows_v, sem):
        wid = lax.axis_index("s") * _NC + lax.axis_index("c")
        base = wid * per_w

        def body(j, carry):
            off = base + j * B
            pltpu.sync_copy(idx_hbm.at[pl.ds(off, B)], idx_v)
            pltpu.async_copy(table_hbm.at[idx_v], rows_v, sem).wait()
            pltpu.sync_copy(rows_v, out_hbm.at[pl.ds(off, B)])
            return carry

        lax.fori_loop(0, nch, body, 0)

    return k(table, idx)


def _sc_scatter_add(vals, idx, R, B):
    M, D = vals.shape
    per_w = M // _NW
    nch = per_w // B
    assert M % _NW == 0 and per_w % B == 0 and B % 8 == 0, (M, B)
    mesh = plsc.VectorSubcoreMesh(core_axis_name="c", subcore_axis_name="s")

    @functools.partial(
        pl.kernel, mesh=mesh,
        out_type=jax.ShapeDtypeStruct((_NC, R, D), jnp.float32),
        compiler_params=pltpu.CompilerParams(use_tc_tiling_on_sc=False),
        scratch_types=[
            pltpu.VMEM((B,), jnp.int32),
            pltpu.VMEM((B, D), jnp.float32),
            pltpu.VMEM_SHARED((R, D), jnp.float32),
            pltpu.SemaphoreType.DMA,
        ],
    )
    def k(vals_hbm, idx_hbm, zeros_hbm, out_hbm, idx_v, val_v, acc, sem):
        cid = lax.axis_index("c")
        sid = lax.axis_index("s")
        base = (sid * _NC + cid) * per_w

        @pl.when(sid == 0)
        def _():
            pltpu.sync_copy(zeros_hbm, acc)

        plsc.subcore_barrier()

        def body(j, carry):
            off = base + j * B
            pltpu.sync_copy(idx_hbm.at[pl.ds(off, B)], idx_v)
            pltpu.sync_copy(vals_hbm.at[pl.ds(off, B)], val_v)
            pltpu.sync_copy(val_v, acc.at[idx_v], add=True)
            return carry

        lax.fori_loop(0, nch, body, 0)
        plsc.subcore_barrier()

        @pl.when(sid == 0)
        def _():
            pltpu.sync_copy(acc, out_hbm.at[cid])

    out = k(vals, idx, jnp.zeros((R, D), jnp.float32))
    return out[0] + out[1]


def _sc_scatter_add_seq(vals, idx, R, B, DP):
    M, D = vals.shape
    per_w = M // _NW
    nch = per_w // B
    NP = D // DP
    assert D % DP == 0 and M % _NW == 0 and per_w % B == 0 and B % 8 == 0
    mesh = plsc.VectorSubcoreMesh(core_axis_name="c", subcore_axis_name="s")

    @functools.partial(
        pl.kernel, mesh=mesh,
        out_type=jax.ShapeDtypeStruct((_NC, NP, R, DP), jnp.float32),
        compiler_params=pltpu.CompilerParams(use_tc_tiling_on_sc=False),
        scratch_types=[
            pltpu.VMEM((B,), jnp.int32),
            pltpu.VMEM((B, DP), jnp.float32),
            pltpu.VMEM_SHARED((R, DP), jnp.float32),
            pltpu.SemaphoreType.DMA,
        ],
    )
    def k(vals_hbm, idx_hbm, zeros_hbm, out_hbm, idx_v, val_v, acc, sem):
        cid = lax.axis_index("c")
        sid = lax.axis_index("s")
        base = (sid * _NC + cid) * per_w
        for p in range(NP):
            @pl.when(sid == 0)
            def _():
                pltpu.sync_copy(zeros_hbm, acc)

            plsc.subcore_barrier()

            def body(j, carry):
                off = base + j * B
                pltpu.sync_copy(idx_hbm.at[pl.ds(off, B)], idx_v)
                pltpu.sync_copy(
                    vals_hbm.at[pl.ds(off, B), pl.ds(DP * p, DP)], val_v)
                pltpu.sync_copy(val_v, acc.at[idx_v], add=True)
                return carry

            lax.fori_loop(0, nch, body, 0)
            plsc.subcore_barrier()

            @pl.when(sid == 0)
            def _():
                pltpu.sync_copy(acc, out_hbm.at[cid, p])

    out = k(vals, idx, jnp.zeros((R, DP), jnp.float32))
    s = out[0] + out[1]
    return jnp.concatenate([s[p] for p in range(NP)], axis=1)


def kernel(X, W_w, W_b, att_v_user, att_v_poi, att_v_class, att_v_time,
           att_e_friend, att_e_visit, att_e_occurrence, att_e_self,
           vertex, edges,
           E_class_index_0, E_class_index_1, E_class_index_2, E_class_index_3, E_class_index,
           V_class_index_0, V_class_index_1, V_class_index_2, V_class_index_3, V_class_index):
    att_e = jnp.stack([att_e_friend[0], att_e_visit[0], att_e_occurrence[0], att_e_self[0]])
    att_v = jnp.stack([att_v_user[0], att_v_poi[0], att_v_class[0], att_v_time[0]])
    A_e = _build_A16(att_e)
    A_v = _build_A16(att_v)

    X0, SvE = _proj(X, W_w.T, W_b, A_e)
    SvE = SvE.reshape(N * 4, 16)

    cls_e = E_class_index[:, 0] // Q
    cls_v = V_class_index[:, 0] // Q
    rE = vertex * 4 + cls_e
    rV = edges * 4 + cls_v

    s = _sc_gather(SvE, rE, B=5000)
    g = jnp.exp(jnp.where(s >= 0, s, 0.2 * s))
    denomE = _sc_scatter_add(g[:, :H].copy(), edges, EDGE_NUM, B=5000)
    beta = g[:, :H] / (_sc_gather(denomE, edges, B=5000) + 1e-16)
    beta64 = jnp.broadcast_to(beta[:, :, None], (NNZ, H, C)).reshape(NNZ, H * C)
    Xve = _sc_gather(X0, vertex)
    Xe = _sc_scatter_add_seq(Xve * beta64, edges, EDGE_NUM, B=1000, DP=32)

    SeV = (Xe @ A_v).reshape(EDGE_NUM * 4, 16)
    s2 = _sc_gather(SeV, rV, B=5000)
    a = jnp.exp(jnp.where(s2 >= 0, s2, 0.2 * s2))
    denomV = _sc_scatter_add(a[:, :H].copy(), vertex, N, B=5000)
    alpha = a[:, :H] / (_sc_gather(denomV, vertex, B=5000) + 1e-16)
    alpha64 = jnp.broadcast_to(alpha[:, :, None], (NNZ, H, C)).reshape(NNZ, H * C)
    Y = _sc_gather(Xe, edges) * alpha64
    Xv = _sc_scatter_add_seq(Y, vertex, N, B=1000, DP=16)
    return jax.nn.relu(Xv)


# speedup vs baseline: 124.5373x; 3.4261x over previous
import functools

import jax
import jax.numpy as jnp
import numpy as np
from jax import lax
from jax.experimental import pallas as pl
from jax.experimental.pallas import tpu as pltpu
from jax.experimental.pallas import tpu_sc as plsc

N = 50000
NNZ = 800000
EDGE_NUM = 25000
IN = 64
H = 8
C = 8
Q = NNZ // 4

_NC = 2
_NS = 16
_NW = _NC * _NS


def _build_A(att4):
    eyeH = jnp.eye(H, dtype=att4.dtype)
    A = att4[:, :, :, None] * eyeH[None, :, None, :]
    return A.transpose(1, 2, 0, 3).reshape(H * C, 4 * H)


def _expander():
    e = np.zeros((H, 64), np.float32)
    for h in range(H):
        e[h, h * 8:(h + 1) * 8] = 1.0
    return jnp.asarray(e)


def _lrelu_exp(x):
    return jnp.exp(jnp.where(x >= 0, x, 0.2 * x))


def _proj_body(x_ref, w_ref, b_ref, ae_ref, eexp_ref, x0_ref, ua_ref, ub_ref):
    x0 = jnp.dot(x_ref[...], w_ref[...], preferred_element_type=jnp.float32)
    x0 = x0 + b_ref[...]
    x0_ref[...] = x0
    sv = jnp.dot(x0, ae_ref[...], preferred_element_type=jnp.float32)
    g = _lrelu_exp(sv)
    for k in range(4):
        g8 = g[:, k * H:(k + 1) * H]
        y = jnp.dot(g8, eexp_ref[...],
                    preferred_element_type=jnp.float32) * x0
        ua_ref[:, k, :] = jnp.concatenate([y[:, :32], g8], axis=1)
        ub_ref[:, k, :] = jnp.concatenate([y[:, 32:], g8], axis=1)


def _proj(X, Wt, b, A_e, Eexp):
    n = X.shape[0]
    BLK = 1000
    return pl.pallas_call(
        _proj_body,
        grid=(n // BLK,),
        in_specs=[
            pl.BlockSpec((BLK, IN), lambda i: (i, 0)),
            pl.BlockSpec((IN, 64), lambda i: (0, 0)),
            pl.BlockSpec((1, 64), lambda i: (0, 0)),
            pl.BlockSpec((64, 32), lambda i: (0, 0)),
            pl.BlockSpec((H, 64), lambda i: (0, 0)),
        ],
        out_specs=[
            pl.BlockSpec((BLK, 64), lambda i: (i, 0)),
            pl.BlockSpec((BLK, 4, 40), lambda i: (i, 0, 0)),
            pl.BlockSpec((BLK, 4, 40), lambda i: (i, 0, 0)),
        ],
        out_shape=[
            jax.ShapeDtypeStruct((n, 64), jnp.float32),
            jax.ShapeDtypeStruct((n, 4, 40), jnp.float32),
            jax.ShapeDtypeStruct((n, 4, 40), jnp.float32),
        ],
    )(X, Wt, b.reshape(1, 64), A_e, Eexp)


def _edge_tables_body(d_ref, av_ref, eexp_ref, w0_ref, w1_ref, w2_ref, w3_ref,
                      w4_ref):
    d = d_ref[...]
    Ua = d[0, 0] + d[1, 0]
    Ub = d[0, 1] + d[1, 1]
    dil = jnp.dot(Ua[:, 32:40] + 1e-16, eexp_ref[...],
                  preferred_element_type=jnp.float32)
    Xe = jnp.concatenate([Ua[:, :32], Ub[:, :32]], axis=1) / dil
    sv = jnp.dot(Xe, av_ref[...], preferred_element_type=jnp.float32)
    a = _lrelu_exp(sv)
    wrefs = [w0_ref, w1_ref, w2_ref, w3_ref]
    for k in range(4):
        a8 = a[:, k * H:(k + 1) * H]
        y = jnp.dot(a8, eexp_ref[...],
                    preferred_element_type=jnp.float32) * Xe
        for q in range(4):
            wrefs[q][:, k, :] = y[:, 16 * q:16 * q + 16]
        w4_ref[:, k, :] = jnp.concatenate([a8, a8], axis=1)


def _edge_tables(dumpA, A_v, Eexp):
    BLK = 1000
    n = EDGE_NUM
    outs = pl.pallas_call(
        _edge_tables_body,
        grid=(n // BLK,),
        in_specs=[
            pl.BlockSpec((2, 2, BLK, 40), lambda i: (0, 0, i, 0)),
            pl.BlockSpec((64, 32), lambda i: (0, 0)),
            pl.BlockSpec((H, 64), lambda i: (0, 0)),
        ],
        out_specs=[pl.BlockSpec((BLK, 4, 16), lambda i: (i, 0, 0))] * 5,
        out_shape=[jax.ShapeDtypeStruct((n, 4, 16), jnp.float32)] * 5,
    )(dumpA, A_v, Eexp)
    return [o.reshape(4 * n, 16) for o in outs]


def _final_body(d_ref, eexp_ref, out_ref):
    d = d_ref[...]
    den = (d[0, 4] + d[1, 4])[:, :H]
    dil = jnp.dot(den + 1e-16, eexp_ref[...],
                  preferred_element_type=jnp.float32)
    y = jnp.concatenate([d[0, q] + d[1, q] for q in range(4)], axis=1)
    out_ref[...] = jax.nn.relu(y / dil)


def _final(dumpB, Eexp):
    BLK = 1000
    return pl.pallas_call(
        _final_body,
        grid=(N // BLK,),
        in_specs=[
            pl.BlockSpec((2, 5, BLK, 16), lambda i: (0, 0, i, 0)),
            pl.BlockSpec((H, 64), lambda i: (0, 0)),
        ],
        out_specs=pl.BlockSpec((BLK, 64), lambda i: (i, 0)),
        out_shape=jax.ShapeDtypeStruct((N, 64), jnp.float32),
    )(dumpB, Eexp)


def _sc_gather_scatter(tables, gidx, sidx, R, B):
    P = len(tables)
    W = tables[0].shape[1]
    M = gidx.shape[0]
    per_w = M // _NW
    nch = per_w // B
    assert M % _NW == 0 and per_w % B == 0 and B % 8 == 0 and W % 8 == 0
    mesh = plsc.VectorSubcoreMesh(core_axis_name="c", subcore_axis_name="s")

    @functools.partial(
        pl.kernel, mesh=mesh,
        out_type=jax.ShapeDtypeStruct((_NC, P, R, W), jnp.float32),
        compiler_params=pltpu.CompilerParams(use_tc_tiling_on_sc=False),
        scratch_types=[
            pltpu.VMEM((B,), jnp.int32),
            pltpu.VMEM((B,), jnp.int32),
            pltpu.VMEM((B, W), jnp.float32),
            pltpu.VMEM_SHARED((R, W), jnp.float32),
            pltpu.SemaphoreType.DMA,
        ],
    )
    def k(*refs):
        t_hbm = refs[:P]
        gidx_hbm, sidx_hbm, zeros_hbm, out_hbm = refs[P:P + 4]
        gi, si, rows, acc, sem = refs[P + 4:]
        cid = lax.axis_index("c")
        sid = lax.axis_index("s")
        base = (sid * _NC + cid) * per_w
        for p in range(P):
            @pl.when(sid == 0)
            def _():
                pltpu.sync_copy(zeros_hbm, acc)

            plsc.subcore_barrier()

            def body(j, carry):
                off = base + j * B
                pltpu.sync_copy(gidx_hbm.at[pl.ds(off, B)], gi)
                pltpu.sync_copy(sidx_hbm.at[pl.ds(off, B)], si)
                pltpu.async_copy(t_hbm[p].at[gi], rows, sem).wait()
                pltpu.sync_copy(rows, acc.at[si], add=True)
                return carry

            lax.fori_loop(0, nch, body, 0)
            plsc.subcore_barrier()

            @pl.when(sid == 0)
            def _():
                pltpu.sync_copy(acc, out_hbm.at[cid, p])

    return k(*tables, gidx, sidx, jnp.zeros((R, W), jnp.float32))


def kernel(X, W_w, W_b, att_v_user, att_v_poi, att_v_class, att_v_time,
           att_e_friend, att_e_visit, att_e_occurrence, att_e_self,
           vertex, edges,
           E_class_index_0, E_class_index_1, E_class_index_2, E_class_index_3, E_class_index,
           V_class_index_0, V_class_index_1, V_class_index_2, V_class_index_3, V_class_index):
    att_e = jnp.stack([att_e_friend[0], att_e_visit[0], att_e_occurrence[0], att_e_self[0]])
    att_v = jnp.stack([att_v_user[0], att_v_poi[0], att_v_class[0], att_v_time[0]])
    A_e = _build_A(att_e)
    A_v = _build_A(att_v)
    Eexp = _expander()

    X0, UEa, UEb = _proj(X, W_w.T, W_b, A_e, Eexp)
    UEa = UEa.reshape(4 * N, 40)
    UEb = UEb.reshape(4 * N, 40)

    rE = vertex * 4 + E_class_index[:, 0] // Q
    rV = edges * 4 + V_class_index[:, 0] // Q

    dumpA = _sc_gather_scatter([UEa, UEb], rE, edges, EDGE_NUM, B=1000)
    WV = _edge_tables(dumpA, A_v, Eexp)
    dumpB = _sc_gather_scatter(WV, rV, vertex, N, B=1000)
    return _final(dumpB, Eexp)
